# baseline (speedup 1.0000x reference)
import jax
import jax.numpy as jnp
from jax import lax
from jax.experimental import pallas as pl
from jax.experimental.pallas import tpu as pltpu

_EMB_DIM = 256
_NUM_EMB = 1024
_COMMIT = 0.25
_B = 16


def _compute_stage(x_ref, emb_ref, e2_ref, loss_ref, sc_ref):
    xb = x_ref[0]
    xe = lax.dot_general(
        emb_ref[...], xb,
        dimension_numbers=(((0,), (0,)), ((), ())),
        preferred_element_type=jnp.float32,
        precision=lax.Precision.DEFAULT,
    )
    sc_ref[...] = e2_ref[0, :][:, None] - 2.0 * xe
    loss_ref[0, 0] += jnp.sum(xb * xb)


def _drain_stage(sc_ref, hi_ref, lo_ref, quant_ref, idx_ref, loss_ref):
    scores = sc_ref[...]
    hw = scores.shape[1]
    idx = jnp.argmin(scores, axis=0).astype(jnp.int32)
    idx_ref[0, 0, :] = idx

    iota_j = lax.broadcasted_iota(jnp.int16, (_NUM_EMB, hw), 0)
    onehot = (iota_j == idx.astype(jnp.int16)[None, :]).astype(jnp.bfloat16)
    quant = lax.dot_general(
        hi_ref[...], onehot,
        dimension_numbers=(((1,), (0,)), ((), ())),
        preferred_element_type=jnp.float32,
    ) + lax.dot_general(
        lo_ref[...], onehot,
        dimension_numbers=(((1,), (0,)), ((), ())),
        preferred_element_type=jnp.float32,
    )
    quant_ref[0] = quant
    loss_ref[0, 0] += jnp.sum(jnp.min(scores, axis=0))


def _vq_body(x_ref, emb_ref, quant_ref, idx_ref, loss_ref,
             hi_ref, lo_ref, e2_ref, scA_ref, scB_ref):
    i = pl.program_id(0)
    even = i % 2 == 0

    @pl.when(i == 0)
    def _():
        emb = emb_ref[...]
        hi = emb.astype(jnp.bfloat16)
        hi_ref[...] = hi
        lo_ref[...] = (emb - hi.astype(jnp.float32)).astype(jnp.bfloat16)
        e2_ref[0, :] = jnp.sum(emb * emb, axis=0)
        loss_ref[0, 0] = 0.0

    @pl.when(jnp.logical_and(i < _B, even))
    def _():
        _compute_stage(x_ref, emb_ref, e2_ref, loss_ref, scA_ref)

    @pl.when(jnp.logical_and(i < _B, jnp.logical_not(even)))
    def _():
        _compute_stage(x_ref, emb_ref, e2_ref, loss_ref, scB_ref)

    @pl.when(jnp.logical_and(i > 0, even))
    def _():
        _drain_stage(scB_ref, hi_ref, lo_ref, quant_ref, idx_ref, loss_ref)

    @pl.when(jnp.logical_and(i > 0, jnp.logical_not(even)))
    def _():
        _drain_stage(scA_ref, hi_ref, lo_ref, quant_ref, idx_ref, loss_ref)


def kernel(x, embed):
    B, C, H, W = x.shape
    HW = H * W
    x3 = x.reshape(B, C, HW)

    quant, idx3, loss_sum = pl.pallas_call(
        _vq_body,
        grid=(B + 1,),
        in_specs=[
            pl.BlockSpec((1, C, HW), lambda i: (jnp.minimum(i, B - 1), 0, 0)),
            pl.BlockSpec((_EMB_DIM, _NUM_EMB), lambda i: (0, 0)),
        ],
        scratch_shapes=[
            pltpu.VMEM((_EMB_DIM, _NUM_EMB), jnp.bfloat16),
            pltpu.VMEM((_EMB_DIM, _NUM_EMB), jnp.bfloat16),
            pltpu.VMEM((1, _NUM_EMB), jnp.float32),
            pltpu.VMEM((_NUM_EMB, HW), jnp.float32),
            pltpu.VMEM((_NUM_EMB, HW), jnp.float32),
        ],
        out_specs=[
            pl.BlockSpec((1, C, HW), lambda i: (jnp.maximum(i - 1, 0), 0, 0)),
            pl.BlockSpec((1, 1, HW), lambda i: (jnp.maximum(i - 1, 0), 0, 0)),
            pl.BlockSpec((1, 1), lambda i: (0, 0), memory_space=pltpu.SMEM),
        ],
        out_shape=[
            jax.ShapeDtypeStruct((B, C, HW), jnp.float32),
            jax.ShapeDtypeStruct((B, 1, HW), jnp.int32),
            jax.ShapeDtypeStruct((1, 1), jnp.float32),
        ],
    )(x3, embed)

    quantize = quant.reshape(B, C, H, W)
    embed_idx = idx3.reshape(B, H, W)
    loss = loss_sum[0, 0] * (_COMMIT / (B * HW * C))
    return quantize, loss, embed_idx


# speedup vs baseline: 1.1827x; 1.1827x over previous
import jax
import jax.numpy as jnp
from jax import lax
from jax.experimental import pallas as pl
from jax.experimental.pallas import tpu as pltpu

_EMB_DIM = 256
_NUM_EMB = 1024
_COMMIT = 0.25
_PER_STEP = 2


def _vq_body(x_ref, emb_ref, quant_ref, idx_ref, loss_ref,
             hi_ref, lo_ref, e2_ref):
    b = pl.program_id(0)

    @pl.when(b == 0)
    def _():
        emb = emb_ref[...]
        hi = emb.astype(jnp.bfloat16)
        hi_ref[...] = hi
        lo_ref[...] = (emb - hi.astype(jnp.float32)).astype(jnp.bfloat16)
        e2_ref[0, :] = jnp.sum(emb * emb, axis=0)
        loss_ref[0, 0] = 0.0

    for s in range(_PER_STEP):
        xb = x_ref[s]
        T = xb.shape[1]
        xe = lax.dot_general(
            emb_ref[...], xb,
            dimension_numbers=(((0,), (0,)), ((), ())),
            preferred_element_type=jnp.float32,
            precision=lax.Precision.DEFAULT,
        )
        scores = e2_ref[0, :][:, None] - 2.0 * xe

        idx = jnp.argmin(scores, axis=0).astype(jnp.int32)
        idx_ref[s, 0, :] = idx

        iota_j = lax.broadcasted_iota(jnp.int32, (_NUM_EMB, T), 0)
        onehot = (iota_j == idx[None, :]).astype(jnp.bfloat16)
        quant = lax.dot_general(
            hi_ref[...], onehot,
            dimension_numbers=(((1,), (0,)), ((), ())),
            preferred_element_type=jnp.float32,
        ) + lax.dot_general(
            lo_ref[...], onehot,
            dimension_numbers=(((1,), (0,)), ((), ())),
            preferred_element_type=jnp.float32,
        )
        quant_ref[s] = quant

        loss_ref[0, 0] += jnp.sum((quant - xb) ** 2)


def kernel(x, embed):
    B, C, H, W = x.shape
    HW = H * W
    x3 = x.reshape(B, C, HW)
    G = B // _PER_STEP

    quant, idx3, loss_sum = pl.pallas_call(
        _vq_body,
        grid=(G,),
        in_specs=[
            pl.BlockSpec((_PER_STEP, C, HW), lambda i: (i, 0, 0)),
            pl.BlockSpec((_EMB_DIM, _NUM_EMB), lambda i: (0, 0)),
        ],
        scratch_shapes=[
            pltpu.VMEM((_EMB_DIM, _NUM_EMB), jnp.bfloat16),
            pltpu.VMEM((_EMB_DIM, _NUM_EMB), jnp.bfloat16),
            pltpu.VMEM((1, _NUM_EMB), jnp.float32),
        ],
        out_specs=[
            pl.BlockSpec((_PER_STEP, C, HW), lambda i: (i, 0, 0)),
            pl.BlockSpec((_PER_STEP, 1, HW), lambda i: (i, 0, 0)),
            pl.BlockSpec((1, 1), lambda i: (0, 0), memory_space=pltpu.SMEM),
        ],
        out_shape=[
            jax.ShapeDtypeStruct((B, C, HW), jnp.float32),
            jax.ShapeDtypeStruct((B, 1, HW), jnp.int32),
            jax.ShapeDtypeStruct((1, 1), jnp.float32),
        ],
    )(x3, embed)

    quantize = quant.reshape(B, C, H, W)
    embed_idx = idx3.reshape(B, H, W)
    loss = loss_sum[0, 0] * (_COMMIT / (B * HW * C))
    return quantize, loss, embed_idx


# speedup vs baseline: 1.1987x; 1.0136x over previous
import jax
import jax.numpy as jnp
from jax import lax
from jax.experimental import pallas as pl
from jax.experimental.pallas import tpu as pltpu

_EMB_DIM = 256
_NUM_EMB = 1024
_COMMIT = 0.25
_PER_STEP = 4


def _vq_body(x_ref, emb_ref, quant_ref, idx_ref, loss_ref,
             hi_ref, lo_ref, e2_ref):
    b = pl.program_id(0)

    @pl.when(b == 0)
    def _():
        emb = emb_ref[...]
        hi = emb.astype(jnp.bfloat16)
        hi_ref[...] = hi
        lo_ref[...] = (emb - hi.astype(jnp.float32)).astype(jnp.bfloat16)
        e2_ref[0, :] = jnp.sum(emb * emb, axis=0)
        loss_ref[0, 0] = 0.0

    for s in range(_PER_STEP):
        xb = x_ref[s]
        T = xb.shape[1]
        xe = lax.dot_general(
            emb_ref[...], xb,
            dimension_numbers=(((0,), (0,)), ((), ())),
            preferred_element_type=jnp.float32,
            precision=lax.Precision.DEFAULT,
        )
        scores = e2_ref[0, :][:, None] - 2.0 * xe

        idx = jnp.argmin(scores, axis=0).astype(jnp.int32)
        idx_ref[s, 0, :] = idx

        iota_j = lax.broadcasted_iota(jnp.int32, (_NUM_EMB, T), 0)
        onehot = (iota_j == idx[None, :]).astype(jnp.bfloat16)
        quant = lax.dot_general(
            hi_ref[...], onehot,
            dimension_numbers=(((1,), (0,)), ((), ())),
            preferred_element_type=jnp.float32,
        ) + lax.dot_general(
            lo_ref[...], onehot,
            dimension_numbers=(((1,), (0,)), ((), ())),
            preferred_element_type=jnp.float32,
        )
        quant_ref[s] = quant

        loss_ref[0, 0] += jnp.sum((quant - xb) ** 2)


def kernel(x, embed):
    B, C, H, W = x.shape
    HW = H * W
    x3 = x.reshape(B, C, HW)
    G = B // _PER_STEP

    quant, idx3, loss_sum = pl.pallas_call(
        _vq_body,
        grid=(G,),
        in_specs=[
            pl.BlockSpec((_PER_STEP, C, HW), lambda i: (i, 0, 0)),
            pl.BlockSpec((_EMB_DIM, _NUM_EMB), lambda i: (0, 0)),
        ],
        scratch_shapes=[
            pltpu.VMEM((_EMB_DIM, _NUM_EMB), jnp.bfloat16),
            pltpu.VMEM((_EMB_DIM, _NUM_EMB), jnp.bfloat16),
            pltpu.VMEM((1, _NUM_EMB), jnp.float32),
        ],
        out_specs=[
            pl.BlockSpec((_PER_STEP, C, HW), lambda i: (i, 0, 0)),
            pl.BlockSpec((_PER_STEP, 1, HW), lambda i: (i, 0, 0)),
            pl.BlockSpec((1, 1), lambda i: (0, 0), memory_space=pltpu.SMEM),
        ],
        out_shape=[
            jax.ShapeDtypeStruct((B, C, HW), jnp.float32),
            jax.ShapeDtypeStruct((B, 1, HW), jnp.int32),
            jax.ShapeDtypeStruct((1, 1), jnp.float32),
        ],
    )(x3, embed)

    quantize = quant.reshape(B, C, H, W)
    embed_idx = idx3.reshape(B, H, W)
    loss = loss_sum[0, 0] * (_COMMIT / (B * HW * C))
    return quantize, loss, embed_idx
